# baseline (speedup 1.0000x reference)
import functools

import jax
import jax.numpy as jnp
from jax import lax
from jax.experimental import pallas as pl
from jax.experimental.pallas import tpu as pltpu
from jax.experimental.pallas import tpu_sc as plsc

N_E = 1024
E_DIM = 256
BETA = 0.25
SIGMA = 0.1

TOK_BLK = 1024


def _dist_body(zb_ref, emb_ref, idx_ref, minv_ref, sumd_ref, acc_ref):
    i = pl.program_id(0)
    z = zb_ref[0].reshape(E_DIM, TOK_BLK)
    e = emb_ref[...]
    zsq = jnp.sum(z * z, axis=0)
    esq = jnp.sum(e * e, axis=1)
    prod2 = lax.dot_general(e + e, z, (((1,), (0,)), ((), ())),
                            preferred_element_type=jnp.float32)
    d = (esq[:, None] + zsq[None, :]) - prod2
    minv = jnp.min(d, axis=0)
    row = lax.broadcasted_iota(jnp.int32, (N_E, TOK_BLK), 0)
    idx = jnp.min(jnp.where(d == minv[None, :], row, N_E), axis=0)
    idx_ref[i, :] = idx
    minv_ref[i, :] = minv

    @pl.when(i == 0)
    def _():
        acc_ref[0] = 0.0

    acc_ref[0] += jnp.sum(minv)

    @pl.when(i == pl.num_programs(0) - 1)
    def _():
        sumd_ref[0, 0] = acc_ref[0]


def _tc_distance_call(zb, emb_w):
    grid = zb.shape[0]
    return pl.pallas_call(
        _dist_body,
        grid=(grid,),
        in_specs=[
            pl.BlockSpec((1, E_DIM, 32, 32), lambda i: (i, 0, 0, 0)),
            pl.BlockSpec((N_E, E_DIM), lambda i: (0, 0)),
        ],
        out_specs=[
            pl.BlockSpec((grid, TOK_BLK), lambda i: (0, 0)),
            pl.BlockSpec((grid, TOK_BLK), lambda i: (0, 0)),
            pl.BlockSpec(memory_space=pltpu.SMEM),
        ],
        out_shape=[
            jax.ShapeDtypeStruct((grid, TOK_BLK), jnp.int32),
            jax.ShapeDtypeStruct((grid, TOK_BLK), jnp.float32),
            jax.ShapeDtypeStruct((1, 1), jnp.float32),
        ],
        scratch_shapes=[pltpu.SMEM((1,), jnp.float32)],
    )(zb, emb_w)


def _fanout_body(zq_ref, zqt_ref, zf1_ref):
    x = zq_ref[...]
    xt = jnp.transpose(x, (1, 0))
    zqt_ref[0] = xt
    zf1_ref[0] = xt.reshape(TOK_BLK, E_DIM)


def _tc_fanout_call(zq_tok, nb):
    return pl.pallas_call(
        _fanout_body,
        grid=(nb,),
        in_specs=[pl.BlockSpec((TOK_BLK, E_DIM), lambda i: (i, 0))],
        out_specs=[
            pl.BlockSpec((1, E_DIM, TOK_BLK), lambda i: (i, 0, 0)),
            pl.BlockSpec((1, TOK_BLK, E_DIM), lambda i: (i, 0, 0)),
        ],
        out_shape=[
            jax.ShapeDtypeStruct((nb, E_DIM, TOK_BLK), jnp.float32),
            jax.ShapeDtypeStruct((nb, TOK_BLK, E_DIM), jnp.float32),
        ],
    )(zq_tok)


def _codebook_body(n_els, emb_ref, minv_ref, sumd_ref, tmin_ref, cbv_ref,
                   dist_ref, loss_ref):
    e = emb_ref[...]
    esq = jnp.sum(e * e, axis=1)
    p2 = lax.dot_general(e + e, e, (((1,), (1,)), ((), ())),
                         preferred_element_type=jnp.float32)
    d1 = (esq[:, None] + esq[None, :]) - p2
    m1 = jnp.min(d1, axis=0)
    row = lax.broadcasted_iota(jnp.int32, (N_E, N_E), 0)
    am = jnp.min(jnp.where(d1 == m1[None, :], row, N_E), axis=0)
    d1x = jnp.where(row == am[None, :], jnp.float32(jnp.inf), d1)
    m2 = jnp.min(d1x, axis=0)
    tot = jnp.sum(m2)
    mean = tot / N_E
    var = jnp.sum((m2 - mean) ** 2) / (N_E - 1)
    tmin_ref[0, 0] = tot
    cbv_ref[0, 0] = jnp.sqrt(var)
    mv = minv_ref[...]
    nw = dist_ref.shape[1]
    tio = lax.broadcasted_iota(jnp.int32, (TOK_BLK, nw), 0)
    wio = lax.broadcasted_iota(jnp.int32, (TOK_BLK, nw), 1)
    sel = jnp.where(jnp.bitwise_and(tio, nw - 1) == wio,
                    jnp.float32(1.0), jnp.float32(0.0))
    hsum = lax.dot_general(mv * mv, sel, (((1,), (0,)), ((), ())),
                           preferred_element_type=jnp.float32)
    nh = TOK_BLK // nw
    dist_ref[...] = jnp.exp(-(hsum / nh) / (2.0 * SIGMA ** 2))
    loss_ref[0, 0] = (1.0 + BETA) * (sumd_ref[0, 0] / n_els) - tot


def _tc_codebook_call(emb_w, minv_b, sumd):
    nb = minv_b.shape[0]
    n_els = nb * TOK_BLK * E_DIM
    return pl.pallas_call(
        functools.partial(_codebook_body, float(n_els)),
        in_specs=[
            pl.BlockSpec((N_E, E_DIM), lambda: (0, 0)),
            pl.BlockSpec((nb, TOK_BLK), lambda: (0, 0)),
            pl.BlockSpec(memory_space=pltpu.SMEM),
        ],
        out_specs=[
            pl.BlockSpec(memory_space=pltpu.SMEM),
            pl.BlockSpec(memory_space=pltpu.SMEM),
            pl.BlockSpec((nb, 32), lambda: (0, 0)),
            pl.BlockSpec(memory_space=pltpu.SMEM),
        ],
        out_shape=[
            jax.ShapeDtypeStruct((1, 1), jnp.float32),
            jax.ShapeDtypeStruct((1, 1), jnp.float32),
            jax.ShapeDtypeStruct((nb, 32), jnp.float32),
            jax.ShapeDtypeStruct((1, 1), jnp.float32),
        ],
    )(emb_w, minv_b, sumd)


def _sc_gather(emb_w, idx2d):
    info = plsc.get_sparse_core_info()
    nc, ns = info.num_cores, info.num_subcores
    nw = nc * ns
    n_idx_rows = idx2d.shape[0]
    rows_per_w = n_idx_rows // nw
    b_per_w = rows_per_w * 128
    n_tok = n_idx_rows * 128
    mesh = plsc.VectorSubcoreMesh(core_axis_name="c", subcore_axis_name="s")

    @functools.partial(
        pl.kernel,
        out_type=jax.ShapeDtypeStruct((n_tok, E_DIM), jnp.float32),
        mesh=mesh,
        scratch_types=[
            pltpu.VMEM((rows_per_w, 128), jnp.int32),
            pltpu.VMEM((b_per_w, E_DIM), jnp.float32),
            pltpu.SemaphoreType.DMA,
        ],
    )
    def gather_kernel(emb_hbm, idx_hbm, out_hbm, idx_v, rows_v, sem):
        wid = lax.axis_index("s") * nc + lax.axis_index("c")
        pltpu.sync_copy(idx_hbm.at[pl.ds(wid * rows_per_w, rows_per_w)], idx_v)
        copies = []
        for j in range(rows_per_w):
            copies.append(pltpu.async_copy(
                emb_hbm.at[idx_v.at[j]],
                rows_v.at[pl.ds(j * 128, 128)],
                sem,
            ))
        for cp in copies:
            cp.wait()
        pltpu.sync_copy(rows_v, out_hbm.at[pl.ds(wid * b_per_w, b_per_w)])

    return gather_kernel(emb_w, idx2d)


def kernel(z, emb_w):
    b, c, h, w = z.shape
    n_tok = b * h * w

    idx_b, minv_b, sumd = _tc_distance_call(z, emb_w)
    zq_tok = _sc_gather(emb_w, idx_b.reshape(-1, 128))
    tmin, cbv, dist, loss = _tc_codebook_call(emb_w, minv_b, sumd)
    idx_flat = idx_b.reshape(n_tok)
    zqt, z_flattened1 = _tc_fanout_call(zq_tok, b)

    z_q = zqt.reshape(b, c, h, w)
    return (z_q, loss[0, 0], dist,
            (None, None, idx_flat),
            z_flattened1, cbv[0, 0], tmin[0, 0])


# speedup vs baseline: 1.0346x; 1.0346x over previous
import functools

import jax
import jax.numpy as jnp
from jax import lax
from jax.experimental import pallas as pl
from jax.experimental.pallas import tpu as pltpu
from jax.experimental.pallas import tpu_sc as plsc

N_E = 1024
E_DIM = 256
BETA = 0.25
SIGMA = 0.1

TOK_BLK = 1024


def _dist_body(zb_ref, emb_ref, idx_ref, minv_ref, sumd_ref, acc_ref):
    i = pl.program_id(0)
    z = zb_ref[0]
    e = emb_ref[...]
    zsq = jnp.sum(z * z, axis=0)
    esq = jnp.sum(e * e, axis=1)
    prod2 = lax.dot_general(e + e, z, (((1,), (0,)), ((), ())),
                            preferred_element_type=jnp.float32)
    d = (esq[:, None] + zsq[None, :]) - prod2
    minv = jnp.min(d, axis=0)
    row = lax.broadcasted_iota(jnp.int32, (N_E, TOK_BLK), 0)
    idx = jnp.min(jnp.where(d == minv[None, :], row, N_E), axis=0)
    idx_ref[i, :] = idx
    minv_ref[i, :] = minv

    @pl.when(i == 0)
    def _():
        acc_ref[0] = 0.0

    acc_ref[0] += jnp.sum(minv)

    @pl.when(i == pl.num_programs(0) - 1)
    def _():
        sumd_ref[0, 0] = acc_ref[0]


def _tc_distance_call(zb, emb_w):
    grid = zb.shape[0]
    return pl.pallas_call(
        _dist_body,
        grid=(grid,),
        in_specs=[
            pl.BlockSpec((1, E_DIM, TOK_BLK), lambda i: (i, 0, 0)),
            pl.BlockSpec((N_E, E_DIM), lambda i: (0, 0)),
        ],
        out_specs=[
            pl.BlockSpec((grid, TOK_BLK), lambda i: (0, 0)),
            pl.BlockSpec((grid, TOK_BLK), lambda i: (0, 0)),
            pl.BlockSpec(memory_space=pltpu.SMEM),
        ],
        out_shape=[
            jax.ShapeDtypeStruct((grid, TOK_BLK), jnp.int32),
            jax.ShapeDtypeStruct((grid, TOK_BLK), jnp.float32),
            jax.ShapeDtypeStruct((1, 1), jnp.float32),
        ],
        scratch_shapes=[pltpu.SMEM((1,), jnp.float32)],
    )(zb, emb_w)


def _fanout_body(zq_ref, zqt_ref, zf1_ref):
    x = zq_ref[...]
    xt = jnp.transpose(x, (1, 0))
    zqt_ref[0] = xt.reshape(E_DIM, 32, 32)
    zf1_ref[0] = xt.reshape(TOK_BLK, E_DIM)


def _tc_fanout_call(zq_tok, nb):
    return pl.pallas_call(
        _fanout_body,
        grid=(nb,),
        in_specs=[pl.BlockSpec((TOK_BLK, E_DIM), lambda i: (i, 0))],
        out_specs=[
            pl.BlockSpec((1, E_DIM, 32, 32), lambda i: (i, 0, 0, 0)),
            pl.BlockSpec((1, TOK_BLK, E_DIM), lambda i: (i, 0, 0)),
        ],
        out_shape=[
            jax.ShapeDtypeStruct((nb, E_DIM, 32, 32), jnp.float32),
            jax.ShapeDtypeStruct((nb, TOK_BLK, E_DIM), jnp.float32),
        ],
    )(zq_tok)


def _codebook_body(n_els, emb_ref, minv_ref, sumd_ref, tmin_ref, cbv_ref,
                   dist_ref, loss_ref):
    e = emb_ref[...]
    esq = jnp.sum(e * e, axis=1)
    p2 = lax.dot_general(e + e, e, (((1,), (1,)), ((), ())),
                         preferred_element_type=jnp.float32)
    d1 = (esq[:, None] + esq[None, :]) - p2
    m1 = jnp.min(d1, axis=0)
    row = lax.broadcasted_iota(jnp.int32, (N_E, N_E), 0)
    am = jnp.min(jnp.where(d1 == m1[None, :], row, N_E), axis=0)
    d1x = jnp.where(row == am[None, :], jnp.float32(jnp.inf), d1)
    m2 = jnp.min(d1x, axis=0)
    tot = jnp.sum(m2)
    mean = tot / N_E
    var = jnp.sum((m2 - mean) ** 2) / (N_E - 1)
    tmin_ref[0, 0] = tot
    cbv_ref[0, 0] = jnp.sqrt(var)
    mv = minv_ref[...]
    nw = dist_ref.shape[1]
    tio = lax.broadcasted_iota(jnp.int32, (TOK_BLK, nw), 0)
    wio = lax.broadcasted_iota(jnp.int32, (TOK_BLK, nw), 1)
    sel = jnp.where(jnp.bitwise_and(tio, nw - 1) == wio,
                    jnp.float32(1.0), jnp.float32(0.0))
    hsum = lax.dot_general(mv * mv, sel, (((1,), (0,)), ((), ())),
                           preferred_element_type=jnp.float32)
    nh = TOK_BLK // nw
    dist_ref[...] = jnp.exp(-(hsum / nh) / (2.0 * SIGMA ** 2))
    loss_ref[0, 0] = (1.0 + BETA) * (sumd_ref[0, 0] / n_els) - tot


def _tc_codebook_call(emb_w, minv_b, sumd):
    nb = minv_b.shape[0]
    n_els = nb * TOK_BLK * E_DIM
    return pl.pallas_call(
        functools.partial(_codebook_body, float(n_els)),
        in_specs=[
            pl.BlockSpec((N_E, E_DIM), lambda: (0, 0)),
            pl.BlockSpec((nb, TOK_BLK), lambda: (0, 0)),
            pl.BlockSpec(memory_space=pltpu.SMEM),
        ],
        out_specs=[
            pl.BlockSpec(memory_space=pltpu.SMEM),
            pl.BlockSpec(memory_space=pltpu.SMEM),
            pl.BlockSpec((nb, 32), lambda: (0, 0)),
            pl.BlockSpec(memory_space=pltpu.SMEM),
        ],
        out_shape=[
            jax.ShapeDtypeStruct((1, 1), jnp.float32),
            jax.ShapeDtypeStruct((1, 1), jnp.float32),
            jax.ShapeDtypeStruct((nb, 32), jnp.float32),
            jax.ShapeDtypeStruct((1, 1), jnp.float32),
        ],
    )(emb_w, minv_b, sumd)


def _sc_gather(emb_w, idx2d):
    info = plsc.get_sparse_core_info()
    nc, ns = info.num_cores, info.num_subcores
    nw = nc * ns
    n_idx_rows = idx2d.shape[0]
    rows_per_w = n_idx_rows // nw
    b_per_w = rows_per_w * 128
    n_tok = n_idx_rows * 128
    mesh = plsc.VectorSubcoreMesh(core_axis_name="c", subcore_axis_name="s")

    @functools.partial(
        pl.kernel,
        out_type=jax.ShapeDtypeStruct((n_tok, E_DIM), jnp.float32),
        mesh=mesh,
        scratch_types=[
            pltpu.VMEM((rows_per_w, 128), jnp.int32),
            pltpu.VMEM((b_per_w, E_DIM), jnp.float32),
            pltpu.SemaphoreType.DMA,
        ],
    )
    def gather_kernel(emb_hbm, idx_hbm, out_hbm, idx_v, rows_v, sem):
        wid = lax.axis_index("s") * nc + lax.axis_index("c")
        pltpu.sync_copy(idx_hbm.at[pl.ds(wid * rows_per_w, rows_per_w)], idx_v)
        copies = []
        for j in range(rows_per_w):
            copies.append(pltpu.async_copy(
                emb_hbm.at[idx_v.at[j]],
                rows_v.at[pl.ds(j * 128, 128)],
                sem,
            ))
        for cp in copies:
            cp.wait()
        pltpu.sync_copy(rows_v, out_hbm.at[pl.ds(wid * b_per_w, b_per_w)])

    return gather_kernel(emb_w, idx2d)


def kernel(z, emb_w):
    b, c, h, w = z.shape
    zb = z.reshape(b, c, h * w)
    n_tok = b * h * w

    idx_b, minv_b, sumd = _tc_distance_call(zb, emb_w)
    zq_tok = _sc_gather(emb_w, idx_b.reshape(-1, 128))
    tmin, cbv, dist, loss = _tc_codebook_call(emb_w, minv_b, sumd)
    idx_flat = idx_b.reshape(n_tok)
    zqt, z_flattened1 = _tc_fanout_call(zq_tok, b)

    z_q = zqt
    return (z_q, loss[0, 0], dist,
            (None, None, idx_flat),
            z_flattened1, cbv[0, 0], tmin[0, 0])


# speedup vs baseline: 1.2776x; 1.2348x over previous
import functools

import jax
import jax.numpy as jnp
from jax import lax
from jax.experimental import pallas as pl
from jax.experimental.pallas import tpu as pltpu
from jax.experimental.pallas import tpu_sc as plsc

N_E = 1024
E_DIM = 256
BETA = 0.25
SIGMA = 0.1

TOK_BLK = 1024


def _dist_body(zb_ref, emb_ref, idx_ref, minv_ref, sumd_ref, acc_ref):
    i = pl.program_id(0)
    z = zb_ref[0]
    e = emb_ref[...]
    zsq = jnp.sum(z * z, axis=0)
    esq = jnp.sum(e * e, axis=1)
    prod2 = lax.dot_general(e + e, z, (((1,), (0,)), ((), ())),
                            preferred_element_type=jnp.float32)
    d = (esq[:, None] + zsq[None, :]) - prod2
    minv = jnp.min(d, axis=0)
    row = lax.broadcasted_iota(jnp.int32, (N_E, TOK_BLK), 0)
    idx = jnp.min(jnp.where(d == minv[None, :], row, N_E), axis=0)
    idx_ref[i, :] = idx
    minv_ref[i, :] = minv

    @pl.when(i == 0)
    def _():
        acc_ref[0] = 0.0

    acc_ref[0] += jnp.sum(minv)

    @pl.when(i == pl.num_programs(0) - 1)
    def _():
        sumd_ref[0, 0] = acc_ref[0]


def _tc_distance_call(zb, emb_w):
    grid = zb.shape[0]
    return pl.pallas_call(
        _dist_body,
        grid=(grid,),
        in_specs=[
            pl.BlockSpec((1, E_DIM, TOK_BLK), lambda i: (i, 0, 0)),
            pl.BlockSpec((N_E, E_DIM), lambda i: (0, 0)),
        ],
        out_specs=[
            pl.BlockSpec((grid, TOK_BLK), lambda i: (0, 0)),
            pl.BlockSpec((grid, TOK_BLK), lambda i: (0, 0)),
            pl.BlockSpec(memory_space=pltpu.SMEM),
        ],
        out_shape=[
            jax.ShapeDtypeStruct((grid, TOK_BLK), jnp.int32),
            jax.ShapeDtypeStruct((grid, TOK_BLK), jnp.float32),
            jax.ShapeDtypeStruct((1, 1), jnp.float32),
        ],
        scratch_shapes=[pltpu.SMEM((1,), jnp.float32)],
    )(zb, emb_w)


def _fanout_body(zq_ref, zqt_ref, zf1_ref):
    x = zq_ref[...]
    xt = jnp.transpose(x, (1, 0))
    zqt_ref[0] = xt
    zf1_ref[0] = xt.reshape(-1, E_DIM)


C_HALF = 128


def _tc_fanout_call(zq_tok, nb):
    return pl.pallas_call(
        _fanout_body,
        grid=(nb * 2,),
        in_specs=[pl.BlockSpec((TOK_BLK, C_HALF), lambda i: (i // 2, i % 2))],
        out_specs=[
            pl.BlockSpec((1, C_HALF, TOK_BLK), lambda i: (i // 2, i % 2, 0)),
            pl.BlockSpec((1, C_HALF * TOK_BLK // E_DIM, E_DIM),
                         lambda i: (i // 2, i % 2, 0)),
        ],
        out_shape=[
            jax.ShapeDtypeStruct((nb, E_DIM, TOK_BLK), jnp.float32),
            jax.ShapeDtypeStruct((nb, TOK_BLK, E_DIM), jnp.float32),
        ],
    )(zq_tok)


def _codebook_body(n_els, emb_ref, minv_ref, sumd_ref, tmin_ref, cbv_ref,
                   dist_ref, loss_ref):
    e = emb_ref[...]
    esq = jnp.sum(e * e, axis=1)
    p2 = lax.dot_general(e + e, e, (((1,), (1,)), ((), ())),
                         preferred_element_type=jnp.float32)
    d1 = (esq[:, None] + esq[None, :]) - p2
    m1 = jnp.min(d1, axis=0)
    row = lax.broadcasted_iota(jnp.int32, (N_E, N_E), 0)
    am = jnp.min(jnp.where(d1 == m1[None, :], row, N_E), axis=0)
    d1x = jnp.where(row == am[None, :], jnp.float32(jnp.inf), d1)
    m2 = jnp.min(d1x, axis=0)
    tot = jnp.sum(m2)
    mean = tot / N_E
    var = jnp.sum((m2 - mean) ** 2) / (N_E - 1)
    tmin_ref[0, 0] = tot
    cbv_ref[0, 0] = jnp.sqrt(var)
    mv = minv_ref[...]
    nw = dist_ref.shape[1]
    tio = lax.broadcasted_iota(jnp.int32, (TOK_BLK, nw), 0)
    wio = lax.broadcasted_iota(jnp.int32, (TOK_BLK, nw), 1)
    sel = jnp.where(jnp.bitwise_and(tio, nw - 1) == wio,
                    jnp.float32(1.0), jnp.float32(0.0))
    hsum = lax.dot_general(mv * mv, sel, (((1,), (0,)), ((), ())),
                           preferred_element_type=jnp.float32)
    nh = TOK_BLK // nw
    dist_ref[...] = jnp.exp(-(hsum / nh) / (2.0 * SIGMA ** 2))
    loss_ref[0, 0] = (1.0 + BETA) * (sumd_ref[0, 0] / n_els) - tot


def _tc_codebook_call(emb_w, minv_b, sumd):
    nb = minv_b.shape[0]
    n_els = nb * TOK_BLK * E_DIM
    return pl.pallas_call(
        functools.partial(_codebook_body, float(n_els)),
        in_specs=[
            pl.BlockSpec((N_E, E_DIM), lambda: (0, 0)),
            pl.BlockSpec((nb, TOK_BLK), lambda: (0, 0)),
            pl.BlockSpec(memory_space=pltpu.SMEM),
        ],
        out_specs=[
            pl.BlockSpec(memory_space=pltpu.SMEM),
            pl.BlockSpec(memory_space=pltpu.SMEM),
            pl.BlockSpec((nb, 32), lambda: (0, 0)),
            pl.BlockSpec(memory_space=pltpu.SMEM),
        ],
        out_shape=[
            jax.ShapeDtypeStruct((1, 1), jnp.float32),
            jax.ShapeDtypeStruct((1, 1), jnp.float32),
            jax.ShapeDtypeStruct((nb, 32), jnp.float32),
            jax.ShapeDtypeStruct((1, 1), jnp.float32),
        ],
    )(emb_w, minv_b, sumd)


def _sc_gather(emb_w, idx2d):
    info = plsc.get_sparse_core_info()
    nc, ns = info.num_cores, info.num_subcores
    nw = nc * ns
    n_idx_rows = idx2d.shape[0]
    rows_per_w = n_idx_rows // nw
    b_per_w = rows_per_w * 128
    n_tok = n_idx_rows * 128
    mesh = plsc.VectorSubcoreMesh(core_axis_name="c", subcore_axis_name="s")

    @functools.partial(
        pl.kernel,
        out_type=jax.ShapeDtypeStruct((n_tok, E_DIM), jnp.float32),
        mesh=mesh,
        scratch_types=[
            pltpu.VMEM((rows_per_w, 128), jnp.int32),
            pltpu.VMEM((b_per_w, E_DIM), jnp.float32),
            pltpu.SemaphoreType.DMA,
        ],
    )
    def gather_kernel(emb_hbm, idx_hbm, out_hbm, idx_v, rows_v, sem):
        wid = lax.axis_index("s") * nc + lax.axis_index("c")
        pltpu.sync_copy(idx_hbm.at[pl.ds(wid * rows_per_w, rows_per_w)], idx_v)
        copies = []
        for j in range(rows_per_w):
            copies.append(pltpu.async_copy(
                emb_hbm.at[idx_v.at[j]],
                rows_v.at[pl.ds(j * 128, 128)],
                sem,
            ))
        for cp in copies:
            cp.wait()
        pltpu.sync_copy(rows_v, out_hbm.at[pl.ds(wid * b_per_w, b_per_w)])

    return gather_kernel(emb_w, idx2d)


def kernel(z, emb_w):
    b, c, h, w = z.shape
    zb = z.reshape(b, c, h * w)
    n_tok = b * h * w

    idx_b, minv_b, sumd = _tc_distance_call(zb, emb_w)
    zq_tok = _sc_gather(emb_w, idx_b.reshape(-1, 128))
    tmin, cbv, dist, loss = _tc_codebook_call(emb_w, minv_b, sumd)
    idx_flat = idx_b.reshape(n_tok)
    zqt, z_flattened1 = _tc_fanout_call(zq_tok, b)

    z_q = zqt.reshape(b, c, h, w)
    return (z_q, loss[0, 0], dist,
            (None, None, idx_flat),
            z_flattened1, cbv[0, 0], tmin[0, 0])


# speedup vs baseline: 1.3909x; 1.0886x over previous
import functools

import jax
import jax.numpy as jnp
from jax import lax
from jax.experimental import pallas as pl
from jax.experimental.pallas import tpu as pltpu
from jax.experimental.pallas import tpu_sc as plsc

N_E = 1024
E_DIM = 256
BETA = 0.25
SIGMA = 0.1

TOK_BLK = 1024


def _dist_body(zb_ref, emb_ref, idx_ref, minv_ref, sumd_ref, acc_ref):
    i = pl.program_id(0)
    z = zb_ref[0]
    e = emb_ref[...]
    zsq = jnp.sum(z * z, axis=0)
    esq = jnp.sum(e * e, axis=1)
    prod2 = lax.dot_general(e + e, z, (((1,), (0,)), ((), ())),
                            preferred_element_type=jnp.float32)
    d = (esq[:, None] + zsq[None, :]) - prod2
    minv = jnp.min(d, axis=0)
    row = lax.broadcasted_iota(jnp.int32, (N_E, TOK_BLK), 0)
    idx = jnp.min(jnp.where(d == minv[None, :], row, N_E), axis=0)
    for k in range(TOK_BLK // 128):
        idx_ref[i, k, :] = lax.slice(idx, (k * 128,), ((k + 1) * 128,))
    minv_ref[i, :] = minv

    @pl.when(i == 0)
    def _():
        acc_ref[0] = 0.0

    acc_ref[0] += jnp.sum(minv)

    @pl.when(i == pl.num_programs(0) - 1)
    def _():
        sumd_ref[0, 0] = acc_ref[0]


def _tc_distance_call(zb, emb_w):
    grid = zb.shape[0]
    return pl.pallas_call(
        _dist_body,
        grid=(grid,),
        in_specs=[
            pl.BlockSpec((1, E_DIM, TOK_BLK), lambda i: (i, 0, 0)),
            pl.BlockSpec((N_E, E_DIM), lambda i: (0, 0)),
        ],
        out_specs=[
            pl.BlockSpec((grid, TOK_BLK // 128, 128), lambda i: (0, 0, 0)),
            pl.BlockSpec((grid, TOK_BLK), lambda i: (0, 0)),
            pl.BlockSpec(memory_space=pltpu.SMEM),
        ],
        out_shape=[
            jax.ShapeDtypeStruct((grid, TOK_BLK // 128, 128), jnp.int32),
            jax.ShapeDtypeStruct((grid, TOK_BLK), jnp.float32),
            jax.ShapeDtypeStruct((1, 1), jnp.float32),
        ],
        scratch_shapes=[pltpu.SMEM((1,), jnp.float32)],
    )(zb, emb_w)


def _fanout_body(zq_ref, zqt_ref, zf1_ref):
    x = zq_ref[...]
    xt = jnp.transpose(x, (1, 0))
    zqt_ref[0] = xt
    zf1_ref[0] = xt.reshape(TOK_BLK, E_DIM)


def _tc_fanout_call(zq_tok, nb):
    return pl.pallas_call(
        _fanout_body,
        grid=(nb,),
        in_specs=[pl.BlockSpec((TOK_BLK, E_DIM), lambda i: (i, 0))],
        out_specs=[
            pl.BlockSpec((1, E_DIM, TOK_BLK), lambda i: (i, 0, 0)),
            pl.BlockSpec((1, TOK_BLK, E_DIM), lambda i: (i, 0, 0)),
        ],
        out_shape=[
            jax.ShapeDtypeStruct((nb, E_DIM, TOK_BLK), jnp.float32),
            jax.ShapeDtypeStruct((nb, TOK_BLK, E_DIM), jnp.float32),
        ],
    )(zq_tok)


def _codebook_body(n_els, emb_ref, minv_ref, sumd_ref, tmin_ref, cbv_ref,
                   dist_ref, loss_ref):
    e = emb_ref[...]
    esq = jnp.sum(e * e, axis=1)
    p2 = lax.dot_general(e + e, e, (((1,), (1,)), ((), ())),
                         preferred_element_type=jnp.float32)
    d1 = (esq[:, None] + esq[None, :]) - p2
    m1 = jnp.min(d1, axis=0)
    row = lax.broadcasted_iota(jnp.int32, (N_E, N_E), 0)
    am = jnp.min(jnp.where(d1 == m1[None, :], row, N_E), axis=0)
    d1x = jnp.where(row == am[None, :], jnp.float32(jnp.inf), d1)
    m2 = jnp.min(d1x, axis=0)
    tot = jnp.sum(m2)
    mean = tot / N_E
    var = jnp.sum((m2 - mean) ** 2) / (N_E - 1)
    tmin_ref[0, 0] = tot
    cbv_ref[0, 0] = jnp.sqrt(var)
    mv = minv_ref[...]
    nw = dist_ref.shape[1]
    tio = lax.broadcasted_iota(jnp.int32, (TOK_BLK, nw), 0)
    wio = lax.broadcasted_iota(jnp.int32, (TOK_BLK, nw), 1)
    sel = jnp.where(jnp.bitwise_and(tio, nw - 1) == wio,
                    jnp.float32(1.0), jnp.float32(0.0))
    hsum = lax.dot_general(mv * mv, sel, (((1,), (0,)), ((), ())),
                           preferred_element_type=jnp.float32)
    nh = TOK_BLK // nw
    dist_ref[...] = jnp.exp(-(hsum / nh) / (2.0 * SIGMA ** 2))
    loss_ref[0, 0] = (1.0 + BETA) * (sumd_ref[0, 0] / n_els) - tot


def _tc_codebook_call(emb_w, minv_b, sumd):
    nb = minv_b.shape[0]
    n_els = nb * TOK_BLK * E_DIM
    return pl.pallas_call(
        functools.partial(_codebook_body, float(n_els)),
        in_specs=[
            pl.BlockSpec((N_E, E_DIM), lambda: (0, 0)),
            pl.BlockSpec((nb, TOK_BLK), lambda: (0, 0)),
            pl.BlockSpec(memory_space=pltpu.SMEM),
        ],
        out_specs=[
            pl.BlockSpec(memory_space=pltpu.SMEM),
            pl.BlockSpec(memory_space=pltpu.SMEM),
            pl.BlockSpec((nb, 32), lambda: (0, 0)),
            pl.BlockSpec(memory_space=pltpu.SMEM),
        ],
        out_shape=[
            jax.ShapeDtypeStruct((1, 1), jnp.float32),
            jax.ShapeDtypeStruct((1, 1), jnp.float32),
            jax.ShapeDtypeStruct((nb, 32), jnp.float32),
            jax.ShapeDtypeStruct((1, 1), jnp.float32),
        ],
    )(emb_w, minv_b, sumd)


def _sc_gather(emb_w, idx2d):
    info = plsc.get_sparse_core_info()
    nc, ns = info.num_cores, info.num_subcores
    nw = nc * ns
    n_idx_rows = idx2d.shape[0]
    rows_per_w = n_idx_rows // nw
    b_per_w = rows_per_w * 128
    n_tok = n_idx_rows * 128
    mesh = plsc.VectorSubcoreMesh(core_axis_name="c", subcore_axis_name="s")

    @functools.partial(
        pl.kernel,
        out_type=jax.ShapeDtypeStruct((n_tok, E_DIM), jnp.float32),
        mesh=mesh,
        scratch_types=[
            pltpu.VMEM((rows_per_w, 128), jnp.int32),
            pltpu.VMEM((b_per_w, E_DIM), jnp.float32),
            pltpu.SemaphoreType.DMA,
        ],
    )
    def gather_kernel(emb_hbm, idx_hbm, out_hbm, idx_v, rows_v, sem):
        wid = lax.axis_index("s") * nc + lax.axis_index("c")
        pltpu.sync_copy(idx_hbm.at[pl.ds(wid * rows_per_w, rows_per_w)], idx_v)
        copies = []
        for j in range(rows_per_w):
            copies.append(pltpu.async_copy(
                emb_hbm.at[idx_v.at[j]],
                rows_v.at[pl.ds(j * 128, 128)],
                sem,
            ))
        for cp in copies:
            cp.wait()
        pltpu.sync_copy(rows_v, out_hbm.at[pl.ds(wid * b_per_w, b_per_w)])

    return gather_kernel(emb_w, idx2d)


def kernel(z, emb_w):
    b, c, h, w = z.shape
    zb = z.reshape(b, c, h * w)
    n_tok = b * h * w

    idx_b3, minv_b, sumd = _tc_distance_call(zb, emb_w)
    zq_tok = _sc_gather(emb_w, idx_b3.reshape(-1, 128))
    tmin, cbv, dist, loss = _tc_codebook_call(emb_w, minv_b, sumd)
    idx_flat = idx_b3.reshape(n_tok)
    zqt, z_flattened1 = _tc_fanout_call(zq_tok, b)

    z_q = zqt.reshape(b, c, h, w)
    return (z_q, loss[0, 0], dist,
            (None, None, idx_flat),
            z_flattened1, cbv[0, 0], tmin[0, 0])
